# baseline (speedup 1.0000x reference)
import functools

import jax
import jax.numpy as jnp
from jax import lax
from jax.experimental import pallas as pl
from jax.experimental.pallas import tpu as pltpu
from jax.experimental.pallas import tpu_sc as plsc

B = 4096
E = 128
L = 50
LP = 56
NC = 2
NS = 16
NW = NC * NS
RPW = B // NW
LANES = 16


def _tanh(x):
    e = jnp.exp(x * 2.0)
    return 1.0 - 2.0 / (e + 1.0)


def _reduce_block(rbuf, base_row, outb, r):
    for k in range(E // LANES):
        sl = pl.ds(k * LANES, LANES)
        acc0 = rbuf[base_row + 0, sl]
        acc1 = rbuf[base_row + 1, sl]
        for j in range(2, LP, 2):
            acc0 = jnp.maximum(acc0, rbuf[base_row + j, sl])
            acc1 = jnp.maximum(acc1, rbuf[base_row + j + 1, sl])
        outb[r, sl] = _tanh(jnp.maximum(acc0, acc1))


def _make_sc_kernel():
    mesh = plsc.VectorSubcoreMesh(core_axis_name="c", subcore_axis_name="s")

    @functools.partial(
        pl.kernel,
        out_type=jax.ShapeDtypeStruct((B, E), jnp.float32),
        mesh=mesh,
        scratch_types=[
            pltpu.VMEM((RPW * LP,), jnp.int32),
            pltpu.VMEM((2 * LP, E), jnp.float32),
            pltpu.VMEM((2 * LP, E), jnp.float32),
            pltpu.VMEM((RPW, E), jnp.float32),
            pltpu.SemaphoreType.DMA,
            pltpu.SemaphoreType.DMA,
        ],
    )
    def sc_kernel(idx_hbm, table_hbm, out_hbm, slab, rows0, rows1, outb,
                  sem0, sem1):
        wid = lax.axis_index("s") * NC + lax.axis_index("c")
        base = wid * RPW

        slab_off = pl.multiple_of(base * LP, 8)
        pltpu.sync_copy(idx_hbm.at[pl.ds(slab_off, RPW * LP)], slab)

        def start(c, rbuf, sem):
            off = pl.multiple_of(c * 2 * LP, 8)
            idxv = slab.at[pl.ds(off, 2 * LP)]
            pltpu.async_copy(table_hbm.at[idxv], rbuf, sem)

        def wait(rbuf, sem):
            pltpu.make_async_copy(
                table_hbm.at[pl.ds(0, 2 * LP)], rbuf, sem).wait()

        def reduce_chunk(rbuf, c):
            _reduce_block(rbuf, 0, outb, 2 * c)
            _reduce_block(rbuf, LP, outb, 2 * c + 1)

        start(0, rows0, sem0)
        start(1, rows1, sem1)

        def body(i, carry):
            a = 2 * i
            wait(rows0, sem0)
            reduce_chunk(rows0, a)
            start(a + 2, rows0, sem0)
            wait(rows1, sem1)
            reduce_chunk(rows1, a + 1)
            start(a + 3, rows1, sem1)
            return carry

        nchunks = RPW // 2
        lax.fori_loop(0, nchunks // 2 - 1, body, 0)

        wait(rows0, sem0)
        reduce_chunk(rows0, nchunks - 2)
        wait(rows1, sem1)
        reduce_chunk(rows1, nchunks - 1)

        pltpu.sync_copy(outb, out_hbm.at[pl.ds(base, RPW)])

    return sc_kernel


_sc_kernel = _make_sc_kernel()


@jax.jit
def kernel(input, table):
    inp = input.astype(jnp.int32)
    inp_p = jnp.concatenate([inp, inp[:, : LP - L]], axis=1)
    idx_flat = inp_p.reshape(-1)
    return _sc_kernel(idx_flat, table)


# speedup vs baseline: 1.0240x; 1.0240x over previous
import functools

import jax
import jax.numpy as jnp
from jax import lax
from jax.experimental import pallas as pl
from jax.experimental.pallas import tpu as pltpu
from jax.experimental.pallas import tpu_sc as plsc

B = 4096
E = 128
L = 50
LP = 56
NC = 2
NS = 16
NW = NC * NS
RPW = B // NW
LANES = 16


def _tanh(x):
    e = jnp.exp(x * 2.0)
    return 1.0 - 2.0 / (e + 1.0)


def _reduce_block(rbuf, base_row, outb, r):
    for k in range(E // LANES):
        sl = pl.ds(k * LANES, LANES)
        acc0 = rbuf[base_row + 0, sl]
        acc1 = rbuf[base_row + 1, sl]
        for j in range(2, LP, 2):
            acc0 = jnp.maximum(acc0, rbuf[base_row + j, sl])
            acc1 = jnp.maximum(acc1, rbuf[base_row + j + 1, sl])
        outb[r, sl] = _tanh(jnp.maximum(acc0, acc1))


def _make_sc_kernel():
    mesh = plsc.VectorSubcoreMesh(core_axis_name="c", subcore_axis_name="s")

    @functools.partial(
        pl.kernel,
        out_type=jax.ShapeDtypeStruct((B, E), jnp.float32),
        mesh=mesh,
        scratch_types=[
            pltpu.VMEM((RPW * LP,), jnp.int32),
            pltpu.VMEM((LP, E), jnp.float32),
            pltpu.VMEM((LP, E), jnp.float32),
            pltpu.VMEM((LP, E), jnp.float32),
            pltpu.VMEM((LP, E), jnp.float32),
            pltpu.VMEM((RPW, E), jnp.float32),
            pltpu.SemaphoreType.DMA,
            pltpu.SemaphoreType.DMA,
            pltpu.SemaphoreType.DMA,
            pltpu.SemaphoreType.DMA,
        ],
    )
    def sc_kernel(idx_hbm, table_hbm, out_hbm, slab, rows0, rows1, rows2,
                  rows3, outb, sem0, sem1, sem2, sem3):
        wid = lax.axis_index("s") * NC + lax.axis_index("c")
        base = wid * RPW
        bufs = (rows0, rows1, rows2, rows3)
        sems = (sem0, sem1, sem2, sem3)
        NBUF = 4

        slab_off = pl.multiple_of(base * LP, 8)
        pltpu.sync_copy(idx_hbm.at[pl.ds(slab_off, RPW * LP)], slab)

        def start(c, rbuf, sem):
            off = pl.multiple_of(c * LP, 8)
            idxv = slab.at[pl.ds(off, LP)]
            pltpu.async_copy(table_hbm.at[idxv], rbuf, sem)

        def wait(rbuf, sem):
            pltpu.make_async_copy(
                table_hbm.at[pl.ds(0, LP)], rbuf, sem).wait()

        for b in range(NBUF):
            start(b, bufs[b], sems[b])

        def body(i, carry):
            a = NBUF * i
            for b in range(NBUF):
                wait(bufs[b], sems[b])
                _reduce_block(bufs[b], 0, outb, a + b)
                start(a + b + NBUF, bufs[b], sems[b])
            return carry

        lax.fori_loop(0, RPW // NBUF - 1, body, 0)

        for b in range(NBUF):
            wait(bufs[b], sems[b])
            _reduce_block(bufs[b], 0, outb, RPW - NBUF + b)

        pltpu.sync_copy(outb, out_hbm.at[pl.ds(base, RPW)])

    return sc_kernel


_sc_kernel = _make_sc_kernel()


@jax.jit
def kernel(input, table):
    inp = input.astype(jnp.int32)
    inp_p = jnp.concatenate([inp, inp[:, : LP - L]], axis=1)
    idx_flat = inp_p.reshape(-1)
    return _sc_kernel(idx_flat, table)


# speedup vs baseline: 1.2955x; 1.2652x over previous
import functools

import jax
import jax.numpy as jnp
from jax import lax
from jax.experimental import pallas as pl
from jax.experimental.pallas import tpu as pltpu
from jax.experimental.pallas import tpu_sc as plsc

B = 4096
E = 128
L = 50
LP = 56
NC = 2
NS = 16
NW = NC * NS
RPW = B // NW
LANES = 16


def _tanh(x):
    e = jnp.exp(x * 2.0)
    return 1.0 - 2.0 / (e + 1.0)


def _reduce_block(rbuf, base_row, outb, r):
    for k in range(E // LANES):
        sl = pl.ds(k * LANES, LANES)
        acc0 = rbuf[base_row + 0, sl]
        acc1 = rbuf[base_row + 1, sl]
        for j in range(2, LP, 2):
            acc0 = jnp.maximum(acc0, rbuf[base_row + j, sl])
            acc1 = jnp.maximum(acc1, rbuf[base_row + j + 1, sl])
        outb[r, sl] = _tanh(jnp.maximum(acc0, acc1))


def _make_sc_kernel():
    mesh = plsc.VectorSubcoreMesh(core_axis_name="c", subcore_axis_name="s")

    @functools.partial(
        pl.kernel,
        out_type=jax.ShapeDtypeStruct((B, E), jnp.float32),
        mesh=mesh,
        scratch_types=[
            pltpu.VMEM((RPW * LP,), jnp.int32),
            pltpu.VMEM((LP, E), jnp.float32),
            pltpu.VMEM((LP, E), jnp.float32),
            pltpu.VMEM((RPW, E), jnp.float32),
            pltpu.SemaphoreType.DMA,
            pltpu.SemaphoreType.DMA,
        ],
    )
    def sc_kernel(idx_hbm, table_hbm, out_hbm, slab, rows0, rows1, outb,
                  sem0, sem1):
        wid = lax.axis_index("s") * NC + lax.axis_index("c")
        base = wid * RPW

        slab_off = pl.multiple_of(base * LP, 8)
        pltpu.sync_copy(idx_hbm.at[pl.ds(slab_off, RPW * LP)], slab)

        def start(c, rbuf, sem):
            off = pl.multiple_of(c * LP, 8)
            idxv = slab.at[pl.ds(off, LP)]
            pltpu.async_copy(table_hbm.at[idxv], rbuf, sem)

        def wait(rbuf, sem):
            pltpu.make_async_copy(
                table_hbm.at[pl.ds(0, LP)], rbuf, sem).wait()

        start(0, rows0, sem0)
        start(1, rows1, sem1)

        def body(i, carry):
            a = 2 * i
            wait(rows0, sem0)
            _reduce_block(rows0, 0, outb, a)
            start(a + 2, rows0, sem0)
            wait(rows1, sem1)
            _reduce_block(rows1, 0, outb, a + 1)
            start(a + 3, rows1, sem1)
            return carry

        lax.fori_loop(0, RPW // 2 - 1, body, 0)

        wait(rows0, sem0)
        _reduce_block(rows0, 0, outb, RPW - 2)
        wait(rows1, sem1)
        _reduce_block(rows1, 0, outb, RPW - 1)

        pltpu.sync_copy(outb, out_hbm.at[pl.ds(base, RPW)])

    return sc_kernel


_sc_kernel = _make_sc_kernel()


@jax.jit
def kernel(input, table):
    inp = input.astype(jnp.int32)
    inp_p = jnp.concatenate([inp, inp[:, : LP - L]], axis=1)
    idx_flat = inp_p.reshape(-1)
    return _sc_kernel(idx_flat, table)


# speedup vs baseline: 1.3608x; 1.0504x over previous
import functools

import jax
import jax.numpy as jnp
from jax import lax
from jax.experimental import pallas as pl
from jax.experimental.pallas import tpu as pltpu
from jax.experimental.pallas import tpu_sc as plsc

B = 4096
E = 128
L = 50
NC = 2
NS = 16
NW = NC * NS
RPW = B // NW
LANES = 16


def _tanh(x):
    e = jnp.exp(x * 2.0)
    return 1.0 - 2.0 / (e + 1.0)


def _reduce_block(rbuf, outb, r):
    for k in range(E // LANES):
        sl = pl.ds(k * LANES, LANES)
        acc0 = rbuf[0, sl]
        acc1 = rbuf[1, sl]
        for j in range(2, L, 2):
            acc0 = jnp.maximum(acc0, rbuf[j, sl])
            acc1 = jnp.maximum(acc1, rbuf[j + 1, sl])
        outb[r, sl] = _tanh(jnp.maximum(acc0, acc1))


def _make_sc_kernel():
    mesh = plsc.VectorSubcoreMesh(core_axis_name="c", subcore_axis_name="s")

    @functools.partial(
        pl.kernel,
        out_type=jax.ShapeDtypeStruct((B, E), jnp.float32),
        mesh=mesh,
        scratch_types=[
            pltpu.VMEM((RPW, L), jnp.int32),
            pltpu.VMEM((L, E), jnp.float32),
            pltpu.VMEM((L, E), jnp.float32),
            pltpu.VMEM((RPW, E), jnp.float32),
            pltpu.SemaphoreType.DMA,
            pltpu.SemaphoreType.DMA,
        ],
    )
    def sc_kernel(idx_hbm, table_hbm, out_hbm, slab, rows0, rows1, outb,
                  sem0, sem1):
        wid = lax.axis_index("s") * NC + lax.axis_index("c")
        base = wid * RPW

        pltpu.sync_copy(idx_hbm.at[pl.ds(base, RPW)], slab)

        def start(c, rbuf, sem):
            idxv = slab.at[c]
            pltpu.async_copy(table_hbm.at[idxv], rbuf, sem)

        def wait(rbuf, sem):
            pltpu.make_async_copy(
                table_hbm.at[slab.at[0]], rbuf, sem).wait()

        start(0, rows0, sem0)
        start(1, rows1, sem1)

        def body(i, carry):
            a = 2 * i
            wait(rows0, sem0)
            _reduce_block(rows0, outb, a)
            start(a + 2, rows0, sem0)
            wait(rows1, sem1)
            _reduce_block(rows1, outb, a + 1)
            start(a + 3, rows1, sem1)
            return carry

        lax.fori_loop(0, RPW // 2 - 1, body, 0)

        wait(rows0, sem0)
        _reduce_block(rows0, outb, RPW - 2)
        wait(rows1, sem1)
        _reduce_block(rows1, outb, RPW - 1)

        pltpu.sync_copy(outb, out_hbm.at[pl.ds(base, RPW)])

    return sc_kernel


_sc_kernel = _make_sc_kernel()


@jax.jit
def kernel(input, table):
    return _sc_kernel(input.astype(jnp.int32), table)


# speedup vs baseline: 1.4188x; 1.0426x over previous
import functools

import jax
import jax.numpy as jnp
from jax import lax
from jax.experimental import pallas as pl
from jax.experimental.pallas import tpu as pltpu
from jax.experimental.pallas import tpu_sc as plsc

B = 4096
E = 128
L = 50
NC = 2
NS = 16
NW = NC * NS
RPW = B // NW
LANES = 16


def _tanh(x):
    e = jnp.exp(x * 2.0)
    return 1.0 - 2.0 / (e + 1.0)


def _reduce_block(rbuf, outb, r):
    for k in range(E // LANES):
        sl = pl.ds(k * LANES, LANES)
        acc0 = rbuf[0, sl]
        acc1 = rbuf[1, sl]
        for j in range(2, L, 2):
            acc0 = jnp.maximum(acc0, rbuf[j, sl])
            acc1 = jnp.maximum(acc1, rbuf[j + 1, sl])
        outb[r, sl] = _tanh(jnp.maximum(acc0, acc1))


def _make_sc_kernel():
    mesh = plsc.VectorSubcoreMesh(core_axis_name="c", subcore_axis_name="s")

    @functools.partial(
        pl.kernel,
        out_type=jax.ShapeDtypeStruct((B, E), jnp.float32),
        mesh=mesh,
        scratch_types=[
            pltpu.VMEM((RPW, L), jnp.int32),
            pltpu.VMEM((L, E), jnp.float32),
            pltpu.VMEM((L, E), jnp.float32),
            pltpu.VMEM((RPW, E), jnp.float32),
            pltpu.SemaphoreType.DMA,
            pltpu.SemaphoreType.DMA,
            pltpu.SemaphoreType.DMA,
            pltpu.SemaphoreType.DMA,
        ],
    )
    def sc_kernel(idx_hbm, table_hbm, out_hbm, slab, rows0, rows1, outb,
                  sem0a, sem0b, sem1a, sem1b):
        wid = lax.axis_index("s") * NC + lax.axis_index("c")
        base = wid * RPW
        LH = L // 2

        pltpu.sync_copy(idx_hbm.at[pl.ds(base, RPW)], slab)

        def start(c, rbuf, sa, sb):
            pltpu.async_copy(table_hbm.at[slab.at[c, pl.ds(0, LH)]],
                             rbuf.at[pl.ds(0, LH)], sa)
            pltpu.async_copy(table_hbm.at[slab.at[c, pl.ds(LH, LH)]],
                             rbuf.at[pl.ds(LH, LH)], sb)

        def wait(rbuf, sa, sb):
            pltpu.make_async_copy(
                table_hbm.at[slab.at[0, pl.ds(0, LH)]],
                rbuf.at[pl.ds(0, LH)], sa).wait()
            pltpu.make_async_copy(
                table_hbm.at[slab.at[0, pl.ds(LH, LH)]],
                rbuf.at[pl.ds(LH, LH)], sb).wait()

        start(0, rows0, sem0a, sem0b)
        start(1, rows1, sem1a, sem1b)

        def body(i, carry):
            a = 2 * i
            wait(rows0, sem0a, sem0b)
            _reduce_block(rows0, outb, a)
            start(a + 2, rows0, sem0a, sem0b)
            wait(rows1, sem1a, sem1b)
            _reduce_block(rows1, outb, a + 1)
            start(a + 3, rows1, sem1a, sem1b)
            return carry

        lax.fori_loop(0, RPW // 2 - 1, body, 0)

        wait(rows0, sem0a, sem0b)
        _reduce_block(rows0, outb, RPW - 2)
        wait(rows1, sem1a, sem1b)
        _reduce_block(rows1, outb, RPW - 1)

        pltpu.sync_copy(outb, out_hbm.at[pl.ds(base, RPW)])

    return sc_kernel


_sc_kernel = _make_sc_kernel()


@jax.jit
def kernel(input, table):
    return _sc_kernel(input.astype(jnp.int32), table)
